# baseline (device time: 1167409 ns/iter reference)
import jax
import jax.numpy as jnp
from jax import lax
from jax.experimental import pallas as pl
from jax.experimental.pallas import tpu as pltpu

N_DEV = 32
M = 4096
N = 8192
CH = M // N_DEV
NR = 4
RH = CH // NR
S = 4


def kernel(x, w_mat):

    def body(x_ref, w_ref, out_ref, *scr):
        it = iter(scr)
        cur_ref = next(it)
        wire = [next(it) for _ in range(NR)]
        comm = [next(it) for _ in range(NR)]
        ag = [next(it) for _ in range(NR)]
        amax_ref = next(it)
        ring_send = [next(it) for _ in range(NR)]
        ring_recv = [next(it) for _ in range(NR)]
        ax_send = next(it)
        ax_recv = next(it)
        store_sems = next(it)
        credit = [next(it) for _ in range(NR)]

        p = lax.axis_index("i")
        right = jnp.mod(p + 1, N_DEV)
        left = jnp.mod(p - 1 + N_DEV, N_DEV)

        barrier_sem = pltpu.get_barrier_semaphore()
        for nbr in (left, right):
            pl.semaphore_signal(barrier_sem, inc=1, device_id=(nbr,),
                                device_id_type=pl.DeviceIdType.MESH)
        pl.semaphore_wait(barrier_sem, 2)

        def gemm_chunk(c, dst_ref):
            xc = x_ref[pl.ds(c * CH, CH), :]
            dst_ref[...] = jnp.dot(xc, w_ref[...],
                                   preferred_element_type=jnp.float32)

        def send_to_right(src_ref, dst_ref, send_sems, recv_sems,
                          src_slot, dst_slot):
            r = pltpu.make_async_remote_copy(
                src_ref=src_ref.at[src_slot],
                dst_ref=dst_ref.at[dst_slot],
                send_sem=send_sems.at[src_slot],
                recv_sem=recv_sems.at[dst_slot],
                device_id=(right,),
                device_id_type=pl.DeviceIdType.MESH,
            )
            r.start()
            return r

        def credit_to(sem):
            pl.semaphore_signal(sem, inc=1, device_id=(left,),
                                device_id_type=pl.DeviceIdType.MESH)

        gemm_chunk(p, cur_ref.at[0])
        for r in range(NR):
            wire[r][0] = cur_ref[0, pl.ds(r * RH, RH), :].astype(jnp.bfloat16)
        rd = [send_to_right(wire[r], comm[r], ring_send[r], ring_recv[r], 0, 0)
              for r in range(NR)]
        prev = [None] * NR
        for s in range(N_DEV - 1):
            b = (s + 1) % 2
            slot = s % S
            gemm_chunk(jnp.mod(p - s - 1, N_DEV), cur_ref.at[b])
            for r in range(NR):
                rows = pl.ds(r * RH, RH)
                rd[r].wait_recv()
                cur_ref[b, rows, :] = (
                    cur_ref[b, rows, :] + comm[r][slot].astype(jnp.float32))
                if prev[r] is not None:
                    prev[r].wait_send()
                wire[r][b] = cur_ref[b, rows, :].astype(jnp.bfloat16)
                if s <= (N_DEV - 2) - S:
                    credit_to(credit[r])
                prev[r] = rd[r]
                if s < N_DEV - 2:
                    if s + 1 >= S:
                        pl.semaphore_wait(credit[r], 1)
                    rd[r] = send_to_right(wire[r], comm[r],
                                          ring_send[r], ring_recv[r],
                                          b, (s + 1) % S)
        for r in range(NR):
            prev[r].wait_send()

        red = cur_ref[(N_DEV - 1) % 2]
        red = jnp.maximum(red, 0.0)

        amax_ref[5] = jnp.full((8, 128), jnp.max(red), dtype=jnp.float32)
        for k in range(5):
            partner = jnp.bitwise_xor(p, 1 << k)
            rdma = pltpu.make_async_remote_copy(
                src_ref=amax_ref.at[5],
                dst_ref=amax_ref.at[k],
                send_sem=ax_send.at[k],
                recv_sem=ax_recv.at[k],
                device_id=(partner,),
                device_id_type=pl.DeviceIdType.MESH,
            )
            rdma.start()
            rdma.wait()
            amax_ref[5] = jnp.maximum(amax_ref[5], amax_ref[k])
        amax = amax_ref[5, 0, 0]

        scale = amax / 127.0
        q = jnp.clip(jnp.round(red / scale), -127.0, 127.0)
        own = jnp.mod(p + 1, N_DEV)
        for r in range(NR):
            ag[r][0] = q[r * RH:(r + 1) * RH].astype(jnp.int8)
        cur_ref[0] = q * scale
        st0 = pltpu.make_async_copy(cur_ref.at[0],
                                    out_ref.at[pl.ds(own * CH, CH), :],
                                    store_sems.at[0])
        st0.start()
        pending_store = {0: st0}

        rd = [send_to_right(ag[r], ag[r], ring_send[r], ring_recv[r], 0, 1)
              for r in range(NR)]
        for g in range(N_DEV - 1):
            recv_slot = (g + 1) % S
            stage = (g + 1) % 2
            if stage in pending_store:
                pending_store.pop(stage).wait()
            for r in range(NR):
                rows = pl.ds(r * RH, RH)
                rd[r].wait_recv()
                rd[r].wait_send()
                cur_ref[stage, rows, :] = (
                    ag[r][recv_slot].astype(jnp.float32) * scale)
                if g <= (N_DEV - 1) - S:
                    credit_to(credit[r])
                if g < N_DEV - 2:
                    if g + 1 >= S - 1:
                        pl.semaphore_wait(credit[r], 1)
                    rd[r] = send_to_right(ag[r], ag[r],
                                          ring_send[r], ring_recv[r],
                                          recv_slot, (g + 2) % S)
            c = jnp.mod(p - g, N_DEV)
            st = pltpu.make_async_copy(cur_ref.at[stage],
                                       out_ref.at[pl.ds(c * CH, CH), :],
                                       store_sems.at[stage])
            st.start()
            pending_store[stage] = st
        for st in pending_store.values():
            st.wait()

    scratch = [pltpu.VMEM((2, CH, N), jnp.float32)]
    scratch += [pltpu.VMEM((2, RH, N), jnp.bfloat16) for _ in range(NR)]
    scratch += [pltpu.VMEM((S, RH, N), jnp.bfloat16) for _ in range(NR)]
    scratch += [pltpu.VMEM((S, RH, N), jnp.int8) for _ in range(NR)]
    scratch += [pltpu.VMEM((8, 8, 128), jnp.float32)]
    scratch += [pltpu.SemaphoreType.DMA((S,)) for _ in range(2 * NR)]
    scratch += [pltpu.SemaphoreType.DMA((5,)) for _ in range(2)]
    scratch += [pltpu.SemaphoreType.DMA((2,))]
    scratch += [pltpu.SemaphoreType.REGULAR for _ in range(NR)]

    return pl.pallas_call(
        body,
        out_shape=jax.ShapeDtypeStruct((M, N), jnp.float32),
        in_specs=[pl.BlockSpec(memory_space=pltpu.VMEM),
                  pl.BlockSpec(memory_space=pltpu.VMEM)],
        out_specs=pl.BlockSpec(memory_space=pl.ANY),
        scratch_shapes=scratch,
        compiler_params=pltpu.CompilerParams(
            collective_id=0, vmem_limit_bytes=100 * 1024 * 1024),
    )(x, w_mat)


# device time: 1166353 ns/iter; 1.0009x vs baseline; 1.0009x over previous
import jax
import jax.numpy as jnp
from jax import lax
from jax.experimental import pallas as pl
from jax.experimental.pallas import tpu as pltpu

N_DEV = 32
M = 4096
N = 8192
CH = M // N_DEV
HT = CH // 2
S = 4


def kernel(x, w_mat):

    def body(x_ref, w_ref, out_ref,
             cur_ref,
             wireA, wireB,
             commA, commB,
             agA, agB,
             amax_ref,
             rsA_send, rsA_recv, rsB_send, rsB_recv,
             agA_send, agA_recv, agB_send, agB_recv,
             ax_send, ax_recv,
             store_sems,
             creditA, creditB):
        p = lax.axis_index("i")
        right = jnp.mod(p + 1, N_DEV)
        left = jnp.mod(p - 1 + N_DEV, N_DEV)

        barrier_sem = pltpu.get_barrier_semaphore()
        for nbr in (left, right):
            pl.semaphore_signal(barrier_sem, inc=1, device_id=(nbr,),
                                device_id_type=pl.DeviceIdType.MESH)
        pl.semaphore_wait(barrier_sem, 2)

        def gemm_chunk(c, dst_ref):
            xc = x_ref[pl.ds(c * CH, CH), :]
            dst_ref[...] = jnp.dot(xc, w_ref[...],
                                   preferred_element_type=jnp.float32)

        def rs_send(wire_ref, comm_ref, send_sems, recv_sems, slot_src, slot):
            r = pltpu.make_async_remote_copy(
                src_ref=wire_ref.at[slot_src],
                dst_ref=comm_ref.at[slot],
                send_sem=send_sems.at[slot],
                recv_sem=recv_sems.at[slot],
                device_id=(right,),
                device_id_type=pl.DeviceIdType.MESH,
            )
            r.start()
            return r

        def credit_to(sem):
            pl.semaphore_signal(sem, inc=1, device_id=(left,),
                                device_id_type=pl.DeviceIdType.MESH)

        gemm_chunk(p, cur_ref.at[0])
        wireA[0] = cur_ref[0, pl.ds(0, HT), :].astype(jnp.bfloat16)
        wireB[0] = cur_ref[0, pl.ds(HT, HT), :].astype(jnp.bfloat16)
        rdA = rs_send(wireA, commA, rsA_send, rsA_recv, 0, 0)
        rdB = rs_send(wireB, commB, rsB_send, rsB_recv, 0, 0)
        prevA = prevB = None
        for s in range(N_DEV - 1):
            b = (s + 1) % 2
            slot = s % S
            gemm_chunk(jnp.mod(p - s - 1, N_DEV), cur_ref.at[b])
            rdA.wait_recv()
            cur_ref[b, pl.ds(0, HT), :] = (
                cur_ref[b, pl.ds(0, HT), :] + commA[slot].astype(jnp.float32))
            if prevA is not None:
                prevA.wait_send()
            wireA[b] = cur_ref[b, pl.ds(0, HT), :].astype(jnp.bfloat16)
            if s <= (N_DEV - 2) - S:
                credit_to(creditA)
            prevA = rdA
            if s < N_DEV - 2:
                if s + 1 >= S:
                    pl.semaphore_wait(creditA, 1)
                rdA = rs_send(wireA, commA, rsA_send, rsA_recv,
                              b, (s + 1) % S)
            rdB.wait_recv()
            cur_ref[b, pl.ds(HT, HT), :] = (
                cur_ref[b, pl.ds(HT, HT), :] + commB[slot].astype(jnp.float32))
            if prevB is not None:
                prevB.wait_send()
            wireB[b] = cur_ref[b, pl.ds(HT, HT), :].astype(jnp.bfloat16)
            if s <= (N_DEV - 2) - S:
                credit_to(creditB)
            prevB = rdB
            if s < N_DEV - 2:
                if s + 1 >= S:
                    pl.semaphore_wait(creditB, 1)
                rdB = rs_send(wireB, commB, rsB_send, rsB_recv,
                              b, (s + 1) % S)
        prevA.wait_send()
        prevB.wait_send()

        red = cur_ref[(N_DEV - 1) % 2]
        red = jnp.maximum(red, 0.0)

        amax_ref[5] = jnp.full((8, 128), jnp.max(red), dtype=jnp.float32)
        for k in range(5):
            partner = jnp.bitwise_xor(p, 1 << k)
            rdma = pltpu.make_async_remote_copy(
                src_ref=amax_ref.at[5],
                dst_ref=amax_ref.at[k],
                send_sem=ax_send.at[k],
                recv_sem=ax_recv.at[k],
                device_id=(partner,),
                device_id_type=pl.DeviceIdType.MESH,
            )
            rdma.start()
            rdma.wait()
            amax_ref[5] = jnp.maximum(amax_ref[5], amax_ref[k])
        amax = amax_ref[5, 0, 0]

        scale = amax / 127.0
        q = jnp.clip(jnp.round(red / scale), -127.0, 127.0)
        own = jnp.mod(p + 1, N_DEV)
        agA[0] = q[:HT].astype(jnp.int8)
        agB[0] = q[HT:].astype(jnp.int8)
        cur_ref[0] = q * scale
        st0 = pltpu.make_async_copy(cur_ref.at[0],
                                    out_ref.at[pl.ds(own * CH, CH), :],
                                    store_sems.at[0])
        st0.start()
        pending_store = {0: st0}

        def ag_send(ag_ref, send_sems, recv_sems, src_slot, dst_slot):
            r = pltpu.make_async_remote_copy(
                src_ref=ag_ref.at[src_slot],
                dst_ref=ag_ref.at[dst_slot],
                send_sem=send_sems.at[src_slot],
                recv_sem=recv_sems.at[dst_slot],
                device_id=(right,),
                device_id_type=pl.DeviceIdType.MESH,
            )
            r.start()
            return r

        rdA = ag_send(agA, agA_send, agA_recv, 0, 1)
        rdB = ag_send(agB, agB_send, agB_recv, 0, 1)
        for g in range(N_DEV - 1):
            recv_slot = (g + 1) % S
            stage = (g + 1) % 2
            if stage in pending_store:
                pending_store.pop(stage).wait()
            rdA.wait_recv()
            rdA.wait_send()
            cur_ref[stage, pl.ds(0, HT), :] = (
                agA[recv_slot].astype(jnp.float32) * scale)
            if g <= (N_DEV - 1) - S:
                credit_to(creditA)
            if g < N_DEV - 2:
                if g + 1 >= S - 1:
                    pl.semaphore_wait(creditA, 1)
                rdA = ag_send(agA, agA_send, agA_recv,
                              recv_slot, (g + 2) % S)
            rdB.wait_recv()
            rdB.wait_send()
            cur_ref[stage, pl.ds(HT, HT), :] = (
                agB[recv_slot].astype(jnp.float32) * scale)
            if g <= (N_DEV - 1) - S:
                credit_to(creditB)
            if g < N_DEV - 2:
                if g + 1 >= S - 1:
                    pl.semaphore_wait(creditB, 1)
                rdB = ag_send(agB, agB_send, agB_recv,
                              recv_slot, (g + 2) % S)
            c = jnp.mod(p - g, N_DEV)
            st = pltpu.make_async_copy(cur_ref.at[stage],
                                       out_ref.at[pl.ds(c * CH, CH), :],
                                       store_sems.at[stage])
            st.start()
            pending_store[stage] = st
        for st in pending_store.values():
            st.wait()

    return pl.pallas_call(
        body,
        out_shape=jax.ShapeDtypeStruct((M, N), jnp.float32),
        in_specs=[pl.BlockSpec(memory_space=pltpu.VMEM),
                  pl.BlockSpec(memory_space=pltpu.VMEM)],
        out_specs=pl.BlockSpec(memory_space=pl.ANY),
        scratch_shapes=[
            pltpu.VMEM((2, CH, N), jnp.float32),
            pltpu.VMEM((2, HT, N), jnp.bfloat16),
            pltpu.VMEM((2, HT, N), jnp.bfloat16),
            pltpu.VMEM((S, HT, N), jnp.bfloat16),
            pltpu.VMEM((S, HT, N), jnp.bfloat16),
            pltpu.VMEM((S, HT, N), jnp.int8),
            pltpu.VMEM((S, HT, N), jnp.int8),
            pltpu.VMEM((8, 8, 128), jnp.float32),
            pltpu.SemaphoreType.DMA((S,)),
            pltpu.SemaphoreType.DMA((S,)),
            pltpu.SemaphoreType.DMA((S,)),
            pltpu.SemaphoreType.DMA((S,)),
            pltpu.SemaphoreType.DMA((S,)),
            pltpu.SemaphoreType.DMA((S,)),
            pltpu.SemaphoreType.DMA((S,)),
            pltpu.SemaphoreType.DMA((S,)),
            pltpu.SemaphoreType.DMA((5,)),
            pltpu.SemaphoreType.DMA((5,)),
            pltpu.SemaphoreType.DMA((2,)),
            pltpu.SemaphoreType.REGULAR,
            pltpu.SemaphoreType.REGULAR,
        ],
        compiler_params=pltpu.CompilerParams(
            collective_id=0, vmem_limit_bytes=100 * 1024 * 1024),
    )(x, w_mat)
